# Initial kernel scaffold; baseline (speedup 1.0000x reference)
import math

import jax
import jax.numpy as jnp
from jax.experimental import pallas as pl
from jax.experimental.pallas import tpu as pltpu

D_MODEL = 1024
N_HEADS = 16
HEAD_DIM = D_MODEL // N_HEADS
B = 2
S = 4096
SAMPLED = max(1, min(S, int(5 * math.log(S + 1))))
BS = 512
HIGH = jax.lax.Precision.HIGHEST


def _qkv_body(x_ref, w_ref, b_ref, k_ref, v_ref, sp_ref, vsum_ref):
    s = pl.program_id(1)
    x = x_ref[0]
    qkv = jax.lax.dot_general(x, w_ref[...], (((1,), (0,)), ((), ())),
                              precision=HIGH,
                              preferred_element_type=jnp.float32)
    qkv = qkv + b_ref[...]
    q = qkv[:, :D_MODEL]
    k_ref[0] = qkv[:, D_MODEL:2 * D_MODEL]
    v = qkv[:, 2 * D_MODEL:]
    v_ref[0] = v
    q2 = (q * q).reshape(BS, N_HEADS, HEAD_DIM)
    sp_ref[0] = jnp.sum(q2, axis=2)

    @pl.when(s == 0)
    def _():
        vsum_ref[0] = jnp.zeros((1, D_MODEL), jnp.float32)
    vsum_ref[0] += jnp.sum(v, axis=0, keepdims=True)


def _qkv_stage(x, w_all, b_all):
    grid = (B, S // BS)
    return pl.pallas_call(
        _qkv_body,
        grid=grid,
        in_specs=[
            pl.BlockSpec((1, BS, D_MODEL), lambda b, s: (b, s, 0)),
            pl.BlockSpec((D_MODEL, 3 * D_MODEL), lambda b, s: (0, 0)),
            pl.BlockSpec((1, 3 * D_MODEL), lambda b, s: (0, 0)),
        ],
        out_specs=[
            pl.BlockSpec((1, BS, D_MODEL), lambda b, s: (b, s, 0)),
            pl.BlockSpec((1, BS, D_MODEL), lambda b, s: (b, s, 0)),
            pl.BlockSpec((1, BS, N_HEADS), lambda b, s: (b, s, 0)),
            pl.BlockSpec((1, 1, D_MODEL), lambda b, s: (b, 0, 0)),
        ],
        out_shape=[
            jax.ShapeDtypeStruct((B, S, D_MODEL), jnp.float32),
            jax.ShapeDtypeStruct((B, S, D_MODEL), jnp.float32),
            jax.ShapeDtypeStruct((B, S, N_HEADS), jnp.float32),
            jax.ShapeDtypeStruct((B, 1, D_MODEL), jnp.float32),
        ],
    )(x, w_all, b_all)


def _topk_body(sp_ref, idx_ref):
    vals = sp_ref[...]
    col = jax.lax.broadcasted_iota(jnp.int32, vals.shape, 1)
    neg = jnp.float32(-jnp.inf)
    for t in range(SAMPLED):
        m = jnp.max(vals, axis=1, keepdims=True)
        eq = vals == m
        arg = jnp.min(jnp.where(eq, col, S), axis=1, keepdims=True)
        idx_ref[:, t:t + 1] = arg
        vals = jnp.where(col == arg, neg, vals)


def _topk_stage(sp):
    return pl.pallas_call(
        _topk_body,
        out_shape=jax.ShapeDtypeStruct((B * N_HEADS, SAMPLED), jnp.int32),
    )(sp)


def _attn_body(idx_ref, x_ref, k_ref, v_ref, wq_ref, bq_ref, wo_ref,
               dout_ref):
    scale = HEAD_DIM ** (-0.5)
    rows = [x_ref[0, pl.ds(idx_ref[0, 0, i], 1), :] for i in range(SAMPLED)]
    selx = jnp.concatenate(rows, axis=0)
    selq = jax.lax.dot_general(selx, wq_ref[...], (((1,), (0,)), ((), ())),
                               precision=HIGH,
                               preferred_element_type=jnp.float32)
    selq = selq + bq_ref[...]
    k = k_ref[0]
    v = v_ref[0]
    scores = jax.lax.dot_general(selq, k, (((1,), (1,)), ((), ())),
                                 precision=HIGH,
                                 preferred_element_type=jnp.float32)
    scores = scores * (HEAD_DIM ** (-0.5))
    m = jnp.max(scores, axis=1, keepdims=True)
    p = jnp.exp(scores - m)
    denom = jnp.sum(p, axis=1, keepdims=True)
    ctx = jax.lax.dot_general(p, v, (((1,), (0,)), ((), ())),
                              precision=HIGH,
                              preferred_element_type=jnp.float32)
    ctx = ctx / denom
    mean_v = jnp.mean(v, axis=0, keepdims=True)
    delta = ctx - mean_v
    dout_ref[0, 0] = jax.lax.dot_general(
        delta, wo_ref[...], (((1,), (0,)), ((), ())),
        precision=HIGH, preferred_element_type=jnp.float32)


def _attn_stage(top_idx, x, k, v, wqt, bq2, wot):
    grid = (B, N_HEADS)
    return pl.pallas_call(
        _attn_body,
        grid=grid,
        in_specs=[
            pl.BlockSpec((1, 1, SAMPLED), lambda b, h: (b, h, 0),
                         memory_space=pltpu.SMEM),
            pl.BlockSpec((1, S, D_MODEL), lambda b, h: (b, 0, 0)),
            pl.BlockSpec((1, S, HEAD_DIM), lambda b, h: (b, 0, h)),
            pl.BlockSpec((1, S, HEAD_DIM), lambda b, h: (b, 0, h)),
            pl.BlockSpec((D_MODEL, HEAD_DIM), lambda b, h: (0, h)),
            pl.BlockSpec((1, HEAD_DIM), lambda b, h: (h, 0)),
            pl.BlockSpec((HEAD_DIM, D_MODEL), lambda b, h: (h, 0)),
        ],
        out_specs=pl.BlockSpec((1, 1, SAMPLED, D_MODEL),
                               lambda b, h: (b, h, 0, 0)),
        out_shape=jax.ShapeDtypeStruct((B, N_HEADS, SAMPLED, D_MODEL),
                                       jnp.float32),
    )(top_idx, x, k, v, wqt, bq2, wot)


def _assemble_body(idx_ref, dout_ref, vsum_ref, wo_ref, bo_ref, out_ref):
    mean_flat = vsum_ref[0] * jnp.float32(1.0 / S)
    base = jax.lax.dot_general(mean_flat, wo_ref[...],
                               (((1,), (0,)), ((), ())),
                               precision=HIGH,
                               preferred_element_type=jnp.float32)
    base = base + bo_ref[...]
    out_ref[0] = jnp.broadcast_to(base, (S, D_MODEL))

    def body(i, carry):
        s = idx_ref[0, i]
        out_ref[0, pl.ds(s, 1), :] += dout_ref[0, pl.ds(i, 1), :]
        return carry
    jax.lax.fori_loop(0, N_HEADS * SAMPLED, body, 0)


def _assemble_stage(idx_flat, dout_flat, vsum, wot, bo2):
    return pl.pallas_call(
        _assemble_body,
        grid=(B,),
        in_specs=[
            pl.BlockSpec((1, N_HEADS * SAMPLED), lambda b: (b, 0),
                         memory_space=pltpu.SMEM),
            pl.BlockSpec((1, N_HEADS * SAMPLED, D_MODEL),
                         lambda b: (b, 0, 0)),
            pl.BlockSpec((1, 1, D_MODEL), lambda b: (b, 0, 0)),
            pl.BlockSpec((D_MODEL, D_MODEL), lambda b: (0, 0)),
            pl.BlockSpec((1, D_MODEL), lambda b: (0, 0)),
        ],
        out_specs=pl.BlockSpec((1, S, D_MODEL), lambda b: (b, 0, 0)),
        out_shape=jax.ShapeDtypeStruct((B, S, D_MODEL), jnp.float32),
    )(idx_flat, dout_flat, vsum, wot, bo2)


@jax.jit
def kernel(x, Wq, bq, Wk, bk, Wv, bv, Wo, bo):
    w_all = jnp.concatenate([Wq.T, Wk.T, Wv.T], axis=1)
    b_all = jnp.concatenate([bq, bk, bv]).reshape(1, 3 * D_MODEL)
    wqt = Wq.T
    wot = Wo.T
    bq2 = bq.reshape(N_HEADS, HEAD_DIM)
    bo2 = bo.reshape(1, D_MODEL)

    k, v, sp, vsum = _qkv_stage(x, w_all, b_all)
    sp_t = sp.transpose(0, 2, 1).reshape(B * N_HEADS, S)
    top_idx = _topk_stage(sp_t).reshape(B, N_HEADS, SAMPLED)
    dout = _attn_stage(top_idx, x, k, v, wqt, bq2, wot)
    idx_flat = top_idx.reshape(B, N_HEADS * SAMPLED)
    dout_flat = dout.reshape(B, N_HEADS * SAMPLED, D_MODEL)
    return _assemble_stage(idx_flat, dout_flat, vsum, wot, bo2)


# speedup vs baseline: 2.7517x; 2.7517x over previous
import math

import jax
import jax.numpy as jnp
from jax.experimental import pallas as pl
from jax.experimental.pallas import tpu as pltpu

D_MODEL = 1024
N_HEADS = 16
HEAD_DIM = D_MODEL // N_HEADS
B = 2
S = 4096
SAMPLED = max(1, min(S, int(5 * math.log(S + 1))))
BS = 512
HIGH = jax.lax.Precision.HIGHEST


def _qkv_body(x_ref, w_ref, b_ref, k_ref, v_ref, sp_ref, vsum_ref):
    s = pl.program_id(1)
    x = x_ref[0].astype(jnp.bfloat16)
    qkv = jax.lax.dot_general(x, w_ref[...], (((1,), (0,)), ((), ())),
                              preferred_element_type=jnp.float32)
    qkv = qkv + b_ref[...]
    q = qkv[:, :D_MODEL]
    k_ref[0] = qkv[:, D_MODEL:2 * D_MODEL].astype(jnp.bfloat16)
    v = qkv[:, 2 * D_MODEL:]
    v_ref[0] = v.astype(jnp.bfloat16)
    q2 = (q * q).reshape(BS, N_HEADS, HEAD_DIM)
    sp_ref[0] = jnp.sum(q2, axis=2)

    @pl.when(s == 0)
    def _():
        vsum_ref[0] = jnp.zeros((1, D_MODEL), jnp.float32)
    vsum_ref[0] += jnp.sum(v, axis=0, keepdims=True)


def _qkv_stage(x, w_all, b_all):
    grid = (B, S // BS)
    return pl.pallas_call(
        _qkv_body,
        grid=grid,
        in_specs=[
            pl.BlockSpec((1, BS, D_MODEL), lambda b, s: (b, s, 0)),
            pl.BlockSpec((D_MODEL, 3 * D_MODEL), lambda b, s: (0, 0)),
            pl.BlockSpec((1, 3 * D_MODEL), lambda b, s: (0, 0)),
        ],
        out_specs=[
            pl.BlockSpec((1, BS, D_MODEL), lambda b, s: (b, s, 0)),
            pl.BlockSpec((1, BS, D_MODEL), lambda b, s: (b, s, 0)),
            pl.BlockSpec((1, BS, N_HEADS), lambda b, s: (b, s, 0)),
            pl.BlockSpec((1, 1, D_MODEL), lambda b, s: (b, 0, 0)),
        ],
        out_shape=[
            jax.ShapeDtypeStruct((B, S, D_MODEL), jnp.bfloat16),
            jax.ShapeDtypeStruct((B, S, D_MODEL), jnp.bfloat16),
            jax.ShapeDtypeStruct((B, S, N_HEADS), jnp.float32),
            jax.ShapeDtypeStruct((B, 1, D_MODEL), jnp.float32),
        ],
    )(x, w_all, b_all)


def _topk_body(sp_ref, idx_ref):
    vals = sp_ref[...]
    col = jax.lax.broadcasted_iota(jnp.int32, vals.shape, 1)
    neg = jnp.float32(-jnp.inf)
    for t in range(SAMPLED):
        m = jnp.max(vals, axis=1, keepdims=True)
        eq = vals == m
        arg = jnp.min(jnp.where(eq, col, S), axis=1, keepdims=True)
        idx_ref[:, t:t + 1] = arg
        vals = jnp.where(col == arg, neg, vals)


def _topk_stage(sp):
    return pl.pallas_call(
        _topk_body,
        out_shape=jax.ShapeDtypeStruct((B * N_HEADS, SAMPLED), jnp.int32),
    )(sp)


def _attn_body(idx_ref, x_ref, k_ref, v_ref, wq_ref, bq_ref, wo_ref,
               vsum_ref, dout_ref):
    for j in range(2):
        lo = j * HEAD_DIM
        rows = [x_ref[0, pl.ds(idx_ref[j, 0, i], 1), :]
                for i in range(SAMPLED)]
        selx = jnp.concatenate(rows, axis=0).astype(jnp.bfloat16)
        wq = wq_ref[:, lo:lo + HEAD_DIM]
        selq = jax.lax.dot_general(selx, wq, (((1,), (0,)), ((), ())),
                                   preferred_element_type=jnp.float32)
        selq = selq + bq_ref[0, 0, lo:lo + HEAD_DIM]
        k = k_ref[0][:, lo:lo + HEAD_DIM]
        v = v_ref[0][:, lo:lo + HEAD_DIM]
        scores = jax.lax.dot_general(selq.astype(jnp.bfloat16), k,
                                     (((1,), (1,)), ((), ())),
                                     preferred_element_type=jnp.float32)
        scores = scores * (HEAD_DIM ** (-0.5))
        m = jnp.max(scores, axis=1, keepdims=True)
        p = jnp.exp(scores - m)
        denom = jnp.sum(p, axis=1, keepdims=True)
        ctx = jax.lax.dot_general(p.astype(jnp.bfloat16), v,
                                  (((1,), (0,)), ((), ())),
                                  preferred_element_type=jnp.float32)
        ctx = ctx / denom
        mean_v = vsum_ref[0, 0, lo:lo + HEAD_DIM] * jnp.float32(1.0 / S)
        delta = (ctx - mean_v).astype(jnp.bfloat16)
        wo = wo_ref[pl.ds(lo, HEAD_DIM), :]
        dout_ref[0, j] = jax.lax.dot_general(
            delta, wo, (((1,), (0,)), ((), ())),
            preferred_element_type=jnp.float32)


def _attn_stage(idx3, x, k, v, wqt, bq3, wot, vsum):
    grid = (B, N_HEADS // 2)
    return pl.pallas_call(
        _attn_body,
        grid=grid,
        in_specs=[
            pl.BlockSpec((2, 1, SAMPLED), lambda b, g: (b * (N_HEADS // 2) + g, 0, 0),
                         memory_space=pltpu.SMEM),
            pl.BlockSpec((1, S, D_MODEL), lambda b, g: (b, 0, 0)),
            pl.BlockSpec((1, S, 2 * HEAD_DIM), lambda b, g: (b, 0, g)),
            pl.BlockSpec((1, S, 2 * HEAD_DIM), lambda b, g: (b, 0, g)),
            pl.BlockSpec((D_MODEL, 2 * HEAD_DIM), lambda b, g: (0, g)),
            pl.BlockSpec((1, 1, 2 * HEAD_DIM), lambda b, g: (g, 0, 0)),
            pl.BlockSpec((2 * HEAD_DIM, D_MODEL), lambda b, g: (g, 0)),
            pl.BlockSpec((1, 1, 2 * HEAD_DIM), lambda b, g: (b, 0, g)),
        ],
        out_specs=pl.BlockSpec((1, 2, SAMPLED, D_MODEL),
                               lambda b, g: (b, g, 0, 0)),
        out_shape=jax.ShapeDtypeStruct((B, N_HEADS, SAMPLED, D_MODEL),
                                       jnp.float32),
    )(idx3, x, k, v, wqt, bq3, wot, vsum)


def _assemble_body(idx_ref, dout_ref, vsum_ref, wo_ref, bo_ref, out_ref):
    mean_flat = vsum_ref[0] * jnp.float32(1.0 / S)
    base = jax.lax.dot_general(mean_flat.astype(jnp.bfloat16), wo_ref[...],
                               (((1,), (0,)), ((), ())),
                               preferred_element_type=jnp.float32)
    base = base + bo_ref[...]
    out_ref[0] = jnp.broadcast_to(base, (S, D_MODEL))

    def body(i, carry):
        s = idx_ref[0, 0, i]
        out_ref[0, pl.ds(s, 1), :] += dout_ref[0, pl.ds(i, 1), :]
        return carry
    jax.lax.fori_loop(0, N_HEADS * SAMPLED, body, 0)


def _assemble_stage(idx_flat, dout_flat, vsum, wot, bo2):
    return pl.pallas_call(
        _assemble_body,
        grid=(B,),
        in_specs=[
            pl.BlockSpec((1, 1, N_HEADS * SAMPLED), lambda b: (b, 0, 0),
                         memory_space=pltpu.SMEM),
            pl.BlockSpec((1, N_HEADS * SAMPLED, D_MODEL),
                         lambda b: (b, 0, 0)),
            pl.BlockSpec((1, 1, D_MODEL), lambda b: (b, 0, 0)),
            pl.BlockSpec((D_MODEL, D_MODEL), lambda b: (0, 0)),
            pl.BlockSpec((1, D_MODEL), lambda b: (0, 0)),
        ],
        out_specs=pl.BlockSpec((1, S, D_MODEL), lambda b: (b, 0, 0)),
        out_shape=jax.ShapeDtypeStruct((B, S, D_MODEL), jnp.float32),
    )(idx_flat, dout_flat, vsum, wot, bo2)


@jax.jit
def kernel(x, Wq, bq, Wk, bk, Wv, bv, Wo, bo):
    w_all = jnp.concatenate([Wq.T, Wk.T, Wv.T],
                            axis=1).astype(jnp.bfloat16)
    b_all = jnp.concatenate([bq, bk, bv]).reshape(1, 3 * D_MODEL)
    wqt = Wq.T.astype(jnp.bfloat16)
    wot = Wo.T.astype(jnp.bfloat16)
    bq3 = bq.reshape(N_HEADS // 2, 1, 2 * HEAD_DIM)
    bo2 = bo.reshape(1, D_MODEL)

    k, v, sp, vsum = _qkv_stage(x, w_all, b_all)
    sp_t = sp.transpose(0, 2, 1).reshape(B * N_HEADS, S)
    top_idx = _topk_stage(sp_t)
    idx3 = top_idx.reshape(B * N_HEADS, 1, SAMPLED)
    dout = _attn_stage(idx3, x, k, v, wqt, bq3, wot, vsum)
    idx_flat = top_idx.reshape(B, 1, N_HEADS * SAMPLED)
    dout_flat = dout.reshape(B, N_HEADS * SAMPLED, D_MODEL)
    return _assemble_stage(idx_flat, dout_flat, vsum, wot, bo2)


# speedup vs baseline: 2.8820x; 1.0473x over previous
import math

import jax
import jax.numpy as jnp
from jax.experimental import pallas as pl
from jax.experimental.pallas import tpu as pltpu

D_MODEL = 1024
N_HEADS = 16
HEAD_DIM = D_MODEL // N_HEADS
B = 2
S = 4096
SAMPLED = max(1, min(S, int(5 * math.log(S + 1))))
BS = 512
OC = 512
NSEL = N_HEADS * SAMPLED


def _qkv_body(x_ref, w_ref, b_ref, q_ref, k_ref, v_ref, sp_ref, vsum_ref):
    s = pl.program_id(1)
    x = x_ref[0].astype(jnp.bfloat16)
    qkv = jax.lax.dot_general(x, w_ref[...], (((1,), (0,)), ((), ())),
                              preferred_element_type=jnp.float32)
    qkv = qkv + b_ref[...]
    q = qkv[:, :D_MODEL]
    q_ref[0] = q.astype(jnp.bfloat16)
    k_ref[0] = qkv[:, D_MODEL:2 * D_MODEL].astype(jnp.bfloat16)
    v = qkv[:, 2 * D_MODEL:]
    v_ref[0] = v.astype(jnp.bfloat16)
    q2 = (q * q).reshape(BS, N_HEADS, HEAD_DIM)
    sp_ref[0] = jnp.sum(q2, axis=2)

    @pl.when(s == 0)
    def _():
        vsum_ref[0] = jnp.zeros((1, D_MODEL), jnp.float32)
    vsum_ref[0] += jnp.sum(v, axis=0, keepdims=True)


def _qkv_stage(x, w_all, b_all):
    grid = (B, S // BS)
    return pl.pallas_call(
        _qkv_body,
        grid=grid,
        in_specs=[
            pl.BlockSpec((1, BS, D_MODEL), lambda b, s: (b, s, 0)),
            pl.BlockSpec((D_MODEL, 3 * D_MODEL), lambda b, s: (0, 0)),
            pl.BlockSpec((1, 3 * D_MODEL), lambda b, s: (0, 0)),
        ],
        out_specs=[
            pl.BlockSpec((1, BS, D_MODEL), lambda b, s: (b, s, 0)),
            pl.BlockSpec((1, BS, D_MODEL), lambda b, s: (b, s, 0)),
            pl.BlockSpec((1, BS, D_MODEL), lambda b, s: (b, s, 0)),
            pl.BlockSpec((1, BS, N_HEADS), lambda b, s: (b, s, 0)),
            pl.BlockSpec((1, 1, D_MODEL), lambda b, s: (b, 0, 0)),
        ],
        out_shape=[
            jax.ShapeDtypeStruct((B, S, D_MODEL), jnp.bfloat16),
            jax.ShapeDtypeStruct((B, S, D_MODEL), jnp.bfloat16),
            jax.ShapeDtypeStruct((B, S, D_MODEL), jnp.bfloat16),
            jax.ShapeDtypeStruct((B, S, N_HEADS), jnp.float32),
            jax.ShapeDtypeStruct((B, 1, D_MODEL), jnp.float32),
        ],
    )(x, w_all, b_all)


def _topk_body(sp_ref, idx_ref):
    vals = sp_ref[...]
    col = jax.lax.broadcasted_iota(jnp.int32, vals.shape, 1)
    neg = jnp.float32(-jnp.inf)
    for t in range(SAMPLED):
        m = jnp.max(vals, axis=1, keepdims=True)
        eq = vals == m
        arg = jnp.min(jnp.where(eq, col, S), axis=1, keepdims=True)
        idx_ref[:, t:t + 1] = arg
        vals = jnp.where(col == arg, neg, vals)


def _topk_stage(sp):
    return pl.pallas_call(
        _topk_body,
        out_shape=jax.ShapeDtypeStruct((B * N_HEADS, SAMPLED), jnp.int32),
    )(sp)


def _attn_body(idx_ref, q_ref, k_ref, v_ref, wo_ref, vsum_ref, dout_ref):
    srow = jax.lax.broadcasted_iota(jnp.int32, (S, SAMPLED), 0)
    for j in range(2):
        lo = j * HEAD_DIM
        oht = (srow == idx_ref[j]).astype(jnp.bfloat16)
        qh = q_ref[0][:, lo:lo + HEAD_DIM]
        selq = jax.lax.dot_general(oht, qh, (((0,), (0,)), ((), ())),
                                   preferred_element_type=jnp.float32)
        selq = selq.astype(jnp.bfloat16)
        k = k_ref[0][:, lo:lo + HEAD_DIM]
        v = v_ref[0][:, lo:lo + HEAD_DIM]
        scores = jax.lax.dot_general(selq, k, (((1,), (1,)), ((), ())),
                                     preferred_element_type=jnp.float32)
        scores = scores * (HEAD_DIM ** (-0.5))
        m = jnp.max(scores, axis=1, keepdims=True)
        p = jnp.exp(scores - m)
        denom = jnp.sum(p, axis=1, keepdims=True)
        ctx = jax.lax.dot_general(p.astype(jnp.bfloat16), v,
                                  (((1,), (0,)), ((), ())),
                                  preferred_element_type=jnp.float32)
        ctx = ctx / denom
        mean_v = vsum_ref[0, 0, lo:lo + HEAD_DIM] * jnp.float32(1.0 / S)
        delta = (ctx - mean_v).astype(jnp.bfloat16)
        wo = wo_ref[pl.ds(lo, HEAD_DIM), :]
        dout = jax.lax.dot_general(delta, wo, (((1,), (0,)), ((), ())),
                                   preferred_element_type=jnp.float32)
        dout_ref[0, j] = dout.astype(jnp.bfloat16)


def _attn_stage(idx3, q, k, v, wot, vsum):
    grid = (B, N_HEADS // 2)
    return pl.pallas_call(
        _attn_body,
        grid=grid,
        in_specs=[
            pl.BlockSpec((2, 1, SAMPLED),
                         lambda b, g: (b * (N_HEADS // 2) + g, 0, 0)),
            pl.BlockSpec((1, S, 2 * HEAD_DIM), lambda b, g: (b, 0, g)),
            pl.BlockSpec((1, S, 2 * HEAD_DIM), lambda b, g: (b, 0, g)),
            pl.BlockSpec((1, S, 2 * HEAD_DIM), lambda b, g: (b, 0, g)),
            pl.BlockSpec((2 * HEAD_DIM, D_MODEL), lambda b, g: (g, 0)),
            pl.BlockSpec((1, 1, 2 * HEAD_DIM), lambda b, g: (b, 0, g)),
        ],
        out_specs=pl.BlockSpec((1, 2, SAMPLED, D_MODEL),
                               lambda b, g: (b, g, 0, 0)),
        out_shape=jax.ShapeDtypeStruct((B, N_HEADS, SAMPLED, D_MODEL),
                                       jnp.bfloat16),
    )(idx3, q, k, v, wot, vsum)


def _assemble_body(idxf_ref, dout_ref, vsum_ref, wo_ref, bo_ref, out_ref,
                   base_ref):
    c = pl.program_id(1)

    @pl.when(c == 0)
    def _():
        mean_flat = vsum_ref[0] * jnp.float32(1.0 / S)
        base = jax.lax.dot_general(mean_flat.astype(jnp.bfloat16),
                                   wo_ref[...], (((1,), (0,)), ((), ())),
                                   preferred_element_type=jnp.float32)
        base_ref[...] = base + bo_ref[...]

    row = jax.lax.broadcasted_iota(jnp.int32, (OC, NSEL), 0) + c * OC
    oh = (row == idxf_ref[0]).astype(jnp.bfloat16)
    scat = jax.lax.dot_general(oh, dout_ref[0], (((1,), (0,)), ((), ())),
                               preferred_element_type=jnp.float32)
    out_ref[0] = scat + base_ref[...]


def _assemble_stage(idxf, dout_flat, vsum, wot, bo2):
    grid = (B, S // OC)
    return pl.pallas_call(
        _assemble_body,
        grid=grid,
        in_specs=[
            pl.BlockSpec((1, 1, NSEL), lambda b, c: (b, 0, 0)),
            pl.BlockSpec((1, NSEL, D_MODEL), lambda b, c: (b, 0, 0)),
            pl.BlockSpec((1, 1, D_MODEL), lambda b, c: (b, 0, 0)),
            pl.BlockSpec((D_MODEL, D_MODEL), lambda b, c: (0, 0)),
            pl.BlockSpec((1, D_MODEL), lambda b, c: (0, 0)),
        ],
        out_specs=pl.BlockSpec((1, OC, D_MODEL), lambda b, c: (b, c, 0)),
        out_shape=jax.ShapeDtypeStruct((B, S, D_MODEL), jnp.float32),
        scratch_shapes=[pltpu.VMEM((1, D_MODEL), jnp.float32)],
    )(idxf, dout_flat, vsum, wot, bo2)


@jax.jit
def kernel(x, Wq, bq, Wk, bk, Wv, bv, Wo, bo):
    w_all = jnp.concatenate([Wq.T, Wk.T, Wv.T],
                            axis=1).astype(jnp.bfloat16)
    b_all = jnp.concatenate([bq, bk, bv]).reshape(1, 3 * D_MODEL)
    wot = Wo.T.astype(jnp.bfloat16)
    bo2 = bo.reshape(1, D_MODEL)

    q, k, v, sp, vsum = _qkv_stage(x, w_all, b_all)
    sp_t = sp.transpose(0, 2, 1).reshape(B * N_HEADS, S)
    top_idx = _topk_stage(sp_t)
    idx3 = top_idx.reshape(B * N_HEADS, 1, SAMPLED)
    dout = _attn_stage(idx3, q, k, v, wot, vsum)
    idxf = top_idx.reshape(B, 1, NSEL)
    dout_flat = dout.reshape(B, NSEL, D_MODEL)
    return _assemble_stage(idxf, dout_flat, vsum, wot, bo2)


# speedup vs baseline: 3.0034x; 1.0421x over previous
import math

import jax
import jax.numpy as jnp
from jax.experimental import pallas as pl
from jax.experimental.pallas import tpu as pltpu

D_MODEL = 1024
N_HEADS = 16
HEAD_DIM = D_MODEL // N_HEADS
B = 2
S = 4096
SAMPLED = max(1, min(S, int(5 * math.log(S + 1))))
BS = 512
OC = 512
NSEL = N_HEADS * SAMPLED


def _qkv_body(x_ref, wq_ref, wk_ref, wv_ref, b_ref,
              q_ref, k_ref, v_ref, sp_ref, vsum_ref):
    s = pl.program_id(1)
    x = x_ref[0].astype(jnp.bfloat16)
    tdims = (((1,), (1,)), ((), ()))
    q = jax.lax.dot_general(x, wq_ref[...], tdims,
                            preferred_element_type=jnp.float32)
    q = q + b_ref[0, 0]
    q_ref[0] = q.astype(jnp.bfloat16)
    k = jax.lax.dot_general(x, wk_ref[...], tdims,
                            preferred_element_type=jnp.float32)
    k_ref[0] = (k + b_ref[0, 1]).astype(jnp.bfloat16)
    v = jax.lax.dot_general(x, wv_ref[...], tdims,
                            preferred_element_type=jnp.float32)
    v = v + b_ref[0, 2]
    v_ref[0] = v.astype(jnp.bfloat16)
    q2 = (q * q).reshape(BS, N_HEADS, HEAD_DIM)
    sp = jnp.sum(q2, axis=2)
    sp_ref[0] = sp.T

    @pl.when(s == 0)
    def _():
        vsum_ref[0] = jnp.zeros((1, D_MODEL), jnp.float32)
    vsum_ref[0] += jnp.sum(v, axis=0, keepdims=True)


def _qkv_stage(x, wq, wk, wv, b3):
    grid = (B, S // BS)
    return pl.pallas_call(
        _qkv_body,
        grid=grid,
        in_specs=[
            pl.BlockSpec((1, BS, D_MODEL), lambda b, s: (b, s, 0)),
            pl.BlockSpec((D_MODEL, D_MODEL), lambda b, s: (0, 0)),
            pl.BlockSpec((D_MODEL, D_MODEL), lambda b, s: (0, 0)),
            pl.BlockSpec((D_MODEL, D_MODEL), lambda b, s: (0, 0)),
            pl.BlockSpec((1, 3, D_MODEL), lambda b, s: (0, 0, 0)),
        ],
        out_specs=[
            pl.BlockSpec((1, BS, D_MODEL), lambda b, s: (b, s, 0)),
            pl.BlockSpec((1, BS, D_MODEL), lambda b, s: (b, s, 0)),
            pl.BlockSpec((1, BS, D_MODEL), lambda b, s: (b, s, 0)),
            pl.BlockSpec((1, N_HEADS, BS), lambda b, s: (b, 0, s)),
            pl.BlockSpec((1, 1, D_MODEL), lambda b, s: (b, 0, 0)),
        ],
        out_shape=[
            jax.ShapeDtypeStruct((B, S, D_MODEL), jnp.bfloat16),
            jax.ShapeDtypeStruct((B, S, D_MODEL), jnp.bfloat16),
            jax.ShapeDtypeStruct((B, S, D_MODEL), jnp.bfloat16),
            jax.ShapeDtypeStruct((B, N_HEADS, S), jnp.float32),
            jax.ShapeDtypeStruct((B, 1, D_MODEL), jnp.float32),
        ],
    )(x, wq, wk, wv, b3)


def _topk_body(sp_ref, idx_ref):
    vals = sp_ref[...]
    col = jax.lax.broadcasted_iota(jnp.int32, vals.shape, 1)
    neg = jnp.float32(-jnp.inf)
    for t in range(SAMPLED):
        m = jnp.max(vals, axis=1, keepdims=True)
        eq = vals == m
        arg = jnp.min(jnp.where(eq, col, S), axis=1, keepdims=True)
        idx_ref[:, t:t + 1] = arg
        vals = jnp.where(col == arg, neg, vals)


def _topk_stage(sp):
    return pl.pallas_call(
        _topk_body,
        out_shape=jax.ShapeDtypeStruct((B * N_HEADS, SAMPLED), jnp.int32),
    )(sp)


def _attn_body(idx_ref, q_ref, k_ref, v_ref, wo_ref, vsum_ref, dout_ref):
    srow = jax.lax.broadcasted_iota(jnp.int32, (S, SAMPLED), 0)
    for j in range(2):
        lo = j * HEAD_DIM
        oht = (srow == idx_ref[j]).astype(jnp.bfloat16)
        qh = q_ref[0][:, lo:lo + HEAD_DIM]
        selq = jax.lax.dot_general(oht, qh, (((0,), (0,)), ((), ())),
                                   preferred_element_type=jnp.float32)
        selq = selq.astype(jnp.bfloat16)
        k = k_ref[0][:, lo:lo + HEAD_DIM]
        v = v_ref[0][:, lo:lo + HEAD_DIM]
        scores = jax.lax.dot_general(selq, k, (((1,), (1,)), ((), ())),
                                     preferred_element_type=jnp.float32)
        scores = scores * (HEAD_DIM ** (-0.5))
        m = jnp.max(scores, axis=1, keepdims=True)
        p = jnp.exp(scores - m)
        denom = jnp.sum(p, axis=1, keepdims=True)
        ctx = jax.lax.dot_general(p.astype(jnp.bfloat16), v,
                                  (((1,), (0,)), ((), ())),
                                  preferred_element_type=jnp.float32)
        ctx = ctx / denom
        mean_v = vsum_ref[0, 0, lo:lo + HEAD_DIM] * jnp.float32(1.0 / S)
        delta = (ctx - mean_v).astype(jnp.bfloat16)
        wo = wo_ref[:, lo:lo + HEAD_DIM]
        dout = jax.lax.dot_general(delta, wo, (((1,), (1,)), ((), ())),
                                   preferred_element_type=jnp.float32)
        dout_ref[0, j] = dout.astype(jnp.bfloat16)


def _attn_stage(idx3, q, k, v, wot, vsum):
    grid = (B, N_HEADS // 2)
    return pl.pallas_call(
        _attn_body,
        grid=grid,
        in_specs=[
            pl.BlockSpec((2, 1, SAMPLED),
                         lambda b, g: (b * (N_HEADS // 2) + g, 0, 0)),
            pl.BlockSpec((1, S, 2 * HEAD_DIM), lambda b, g: (b, 0, g)),
            pl.BlockSpec((1, S, 2 * HEAD_DIM), lambda b, g: (b, 0, g)),
            pl.BlockSpec((1, S, 2 * HEAD_DIM), lambda b, g: (b, 0, g)),
            pl.BlockSpec((D_MODEL, 2 * HEAD_DIM), lambda b, g: (0, g)),
            pl.BlockSpec((1, 1, 2 * HEAD_DIM), lambda b, g: (b, 0, g)),
        ],
        out_specs=pl.BlockSpec((1, 2, SAMPLED, D_MODEL),
                               lambda b, g: (b, g, 0, 0)),
        out_shape=jax.ShapeDtypeStruct((B, N_HEADS, SAMPLED, D_MODEL),
                                       jnp.bfloat16),
    )(idx3, q, k, v, wot, vsum)


def _assemble_body(idxf_ref, dout_ref, vsum_ref, wo_ref, bo_ref, out_ref,
                   base_ref):
    c = pl.program_id(1)

    @pl.when(c == 0)
    def _():
        mean_flat = vsum_ref[0] * jnp.float32(1.0 / S)
        base = jax.lax.dot_general(mean_flat.astype(jnp.bfloat16),
                                   wo_ref[...], (((1,), (1,)), ((), ())),
                                   preferred_element_type=jnp.float32)
        base_ref[...] = base + bo_ref[...]

    row = jax.lax.broadcasted_iota(jnp.int32, (OC, NSEL), 0) + c * OC
    oh = (row == idxf_ref[0]).astype(jnp.bfloat16)
    scat = jax.lax.dot_general(oh, dout_ref[0], (((1,), (0,)), ((), ())),
                               preferred_element_type=jnp.float32)
    out_ref[0] = scat + base_ref[...]


def _assemble_stage(idxf, dout_flat, vsum, wot, bo2):
    grid = (B, S // OC)
    return pl.pallas_call(
        _assemble_body,
        grid=grid,
        in_specs=[
            pl.BlockSpec((1, 1, NSEL), lambda b, c: (b, 0, 0)),
            pl.BlockSpec((1, NSEL, D_MODEL), lambda b, c: (b, 0, 0)),
            pl.BlockSpec((1, 1, D_MODEL), lambda b, c: (b, 0, 0)),
            pl.BlockSpec((D_MODEL, D_MODEL), lambda b, c: (0, 0)),
            pl.BlockSpec((1, D_MODEL), lambda b, c: (0, 0)),
        ],
        out_specs=pl.BlockSpec((1, OC, D_MODEL), lambda b, c: (b, c, 0)),
        out_shape=jax.ShapeDtypeStruct((B, S, D_MODEL), jnp.float32),
        scratch_shapes=[pltpu.VMEM((1, D_MODEL), jnp.float32)],
    )(idxf, dout_flat, vsum, wot, bo2)


@jax.jit
def kernel(x, Wq, bq, Wk, bk, Wv, bv, Wo, bo):
    wq_bf = Wq.astype(jnp.bfloat16)
    wk_bf = Wk.astype(jnp.bfloat16)
    wv_bf = Wv.astype(jnp.bfloat16)
    wo_bf = Wo.astype(jnp.bfloat16)
    b3 = jnp.stack([bq, bk, bv]).reshape(1, 3, D_MODEL)
    bo2 = bo.reshape(1, D_MODEL)

    q, k, v, sp, vsum = _qkv_stage(x, wq_bf, wk_bf, wv_bf, b3)
    top_idx = _topk_stage(sp.reshape(B * N_HEADS, S))
    idx3 = top_idx.reshape(B * N_HEADS, 1, SAMPLED)
    dout = _attn_stage(idx3, q, k, v, wo_bf, vsum)
    idxf = top_idx.reshape(B, 1, NSEL)
    dout_flat = dout.reshape(B, NSEL, D_MODEL)
    return _assemble_stage(idxf, dout_flat, vsum, wo_bf, bo2)


# speedup vs baseline: 3.3047x; 1.1003x over previous
import math

import jax
import jax.numpy as jnp
from jax.experimental import pallas as pl
from jax.experimental.pallas import tpu as pltpu

D_MODEL = 1024
N_HEADS = 16
HEAD_DIM = D_MODEL // N_HEADS
B = 2
S = 4096
SAMPLED = max(1, min(S, int(5 * math.log(S + 1))))
BS = 512
OC = 512
NSEL = N_HEADS * SAMPLED


def _qkv_body(x_ref, wq_ref, wk_ref, wv_ref, b_ref,
              q_ref, k_ref, v_ref, sp_ref, vsum_ref):
    s = pl.program_id(1)
    x = x_ref[0].astype(jnp.bfloat16)
    tdims = (((1,), (1,)), ((), ()))
    q = jax.lax.dot_general(x, wq_ref[...], tdims,
                            preferred_element_type=jnp.float32)
    q = q + b_ref[0, 0]
    q_ref[0] = q.astype(jnp.bfloat16)
    k = jax.lax.dot_general(x, wk_ref[...], tdims,
                            preferred_element_type=jnp.float32)
    k_ref[0] = (k + b_ref[0, 1]).astype(jnp.bfloat16)
    v = jax.lax.dot_general(x, wv_ref[...], tdims,
                            preferred_element_type=jnp.float32)
    v = v + b_ref[0, 2]
    v_ref[0] = v.astype(jnp.bfloat16)
    q2 = (q * q).reshape(BS, N_HEADS, HEAD_DIM)
    sp = jnp.sum(q2, axis=2)
    sp_ref[0] = sp.T

    @pl.when(s == 0)
    def _():
        vsum_ref[0] = jnp.zeros((1, D_MODEL), jnp.float32)
    vsum_ref[0] += jnp.sum(v, axis=0, keepdims=True)


def _qkv_stage(x, wq, wk, wv, b3):
    grid = (B, S // BS)
    return pl.pallas_call(
        _qkv_body,
        grid=grid,
        in_specs=[
            pl.BlockSpec((1, BS, D_MODEL), lambda b, s: (b, s, 0)),
            pl.BlockSpec((D_MODEL, D_MODEL), lambda b, s: (0, 0)),
            pl.BlockSpec((D_MODEL, D_MODEL), lambda b, s: (0, 0)),
            pl.BlockSpec((D_MODEL, D_MODEL), lambda b, s: (0, 0)),
            pl.BlockSpec((1, 3, D_MODEL), lambda b, s: (0, 0, 0)),
        ],
        out_specs=[
            pl.BlockSpec((1, BS, D_MODEL), lambda b, s: (b, s, 0)),
            pl.BlockSpec((1, BS, D_MODEL), lambda b, s: (b, s, 0)),
            pl.BlockSpec((1, BS, D_MODEL), lambda b, s: (b, s, 0)),
            pl.BlockSpec((1, N_HEADS, BS), lambda b, s: (b, 0, s)),
            pl.BlockSpec((1, 1, D_MODEL), lambda b, s: (b, 0, 0)),
        ],
        out_shape=[
            jax.ShapeDtypeStruct((B, S, D_MODEL), jnp.bfloat16),
            jax.ShapeDtypeStruct((B, S, D_MODEL), jnp.bfloat16),
            jax.ShapeDtypeStruct((B, S, D_MODEL), jnp.bfloat16),
            jax.ShapeDtypeStruct((B, N_HEADS, S), jnp.float32),
            jax.ShapeDtypeStruct((B, 1, D_MODEL), jnp.float32),
        ],
    )(x, wq, wk, wv, b3)


def _topk_body(sp_ref, idx_ref):
    vals = sp_ref[...]
    col = jax.lax.broadcasted_iota(jnp.int32, vals.shape, 1)
    neg = jnp.float32(-jnp.inf)
    for t in range(SAMPLED):
        m = jnp.max(vals, axis=1, keepdims=True)
        eq = vals == m
        arg = jnp.min(jnp.where(eq, col, S), axis=1, keepdims=True)
        idx_ref[:, t:t + 1] = arg
        vals = jnp.where(col == arg, neg, vals)


def _topk_stage(sp):
    return pl.pallas_call(
        _topk_body,
        out_shape=jax.ShapeDtypeStruct((B * N_HEADS, SAMPLED), jnp.int32),
    )(sp)


HP = 4
CW = HP * HEAD_DIM
RW = HP * SAMPLED


def _attn_body(idx_ref, q_ref, k_ref, v_ref, wo_ref, vsum_ref, dout_ref):
    srow = jax.lax.broadcasted_iota(jnp.int32, (S, RW), 0)
    idx_cat = jnp.concatenate([idx_ref[j] for j in range(HP)], axis=1)
    oht = (srow == idx_cat).astype(jnp.bfloat16)
    rblk = jax.lax.broadcasted_iota(jnp.int32, (RW, CW), 0) // SAMPLED
    cblk = jax.lax.broadcasted_iota(jnp.int32, (RW, CW), 1) // HEAD_DIM
    mask = rblk == cblk
    selqf = jax.lax.dot_general(oht, q_ref[0], (((0,), (0,)), ((), ())),
                                preferred_element_type=jnp.float32)
    selq = jnp.where(mask, selqf, 0.0).astype(jnp.bfloat16)
    scores = jax.lax.dot_general(selq, k_ref[0], (((1,), (1,)), ((), ())),
                                 preferred_element_type=jnp.float32)
    scores = scores * (HEAD_DIM ** (-0.5))
    m = jnp.max(scores, axis=1, keepdims=True)
    p = jnp.exp(scores - m)
    denom = jnp.sum(p, axis=1, keepdims=True)
    ctx = jax.lax.dot_general(p.astype(jnp.bfloat16), v_ref[0],
                              (((1,), (0,)), ((), ())),
                              preferred_element_type=jnp.float32)
    ctx = ctx / denom
    mean4 = vsum_ref[0, 0] * jnp.float32(1.0 / S)
    delta = jnp.where(mask, ctx - mean4, 0.0).astype(jnp.bfloat16)
    dout = jax.lax.dot_general(delta, wo_ref[...], (((1,), (1,)), ((), ())),
                               preferred_element_type=jnp.float32)
    dout = dout.astype(jnp.bfloat16)
    for j in range(HP):
        dout_ref[0, j] = dout[j * SAMPLED:(j + 1) * SAMPLED]


def _attn_stage(idx3, q, k, v, wot, vsum):
    grid = (B, N_HEADS // HP)
    return pl.pallas_call(
        _attn_body,
        grid=grid,
        in_specs=[
            pl.BlockSpec((HP, 1, SAMPLED),
                         lambda b, g: (b * (N_HEADS // HP) + g, 0, 0)),
            pl.BlockSpec((1, S, CW), lambda b, g: (b, 0, g)),
            pl.BlockSpec((1, S, CW), lambda b, g: (b, 0, g)),
            pl.BlockSpec((1, S, CW), lambda b, g: (b, 0, g)),
            pl.BlockSpec((D_MODEL, CW), lambda b, g: (0, g)),
            pl.BlockSpec((1, 1, CW), lambda b, g: (b, 0, g)),
        ],
        out_specs=pl.BlockSpec((1, HP, SAMPLED, D_MODEL),
                               lambda b, g: (b, g, 0, 0)),
        out_shape=jax.ShapeDtypeStruct((B, N_HEADS, SAMPLED, D_MODEL),
                                       jnp.bfloat16),
    )(idx3, q, k, v, wot, vsum)


def _assemble_body(idxf_ref, dout_ref, vsum_ref, wo_ref, bo_ref, out_ref,
                   base_ref):
    c = pl.program_id(1)

    @pl.when(c == 0)
    def _():
        mean_flat = vsum_ref[0] * jnp.float32(1.0 / S)
        base = jax.lax.dot_general(mean_flat.astype(jnp.bfloat16),
                                   wo_ref[...], (((1,), (1,)), ((), ())),
                                   preferred_element_type=jnp.float32)
        base_ref[...] = base + bo_ref[...]

    row = jax.lax.broadcasted_iota(jnp.int32, (OC, NSEL), 0) + c * OC
    oh = (row == idxf_ref[0]).astype(jnp.bfloat16)
    scat = jax.lax.dot_general(oh, dout_ref[0], (((1,), (0,)), ((), ())),
                               preferred_element_type=jnp.float32)
    out_ref[0] = scat + base_ref[...]


def _assemble_stage(idxf, dout_flat, vsum, wot, bo2):
    grid = (B, S // OC)
    return pl.pallas_call(
        _assemble_body,
        grid=grid,
        in_specs=[
            pl.BlockSpec((1, 1, NSEL), lambda b, c: (b, 0, 0)),
            pl.BlockSpec((1, NSEL, D_MODEL), lambda b, c: (b, 0, 0)),
            pl.BlockSpec((1, 1, D_MODEL), lambda b, c: (b, 0, 0)),
            pl.BlockSpec((D_MODEL, D_MODEL), lambda b, c: (0, 0)),
            pl.BlockSpec((1, D_MODEL), lambda b, c: (0, 0)),
        ],
        out_specs=pl.BlockSpec((1, OC, D_MODEL), lambda b, c: (b, c, 0)),
        out_shape=jax.ShapeDtypeStruct((B, S, D_MODEL), jnp.float32),
        scratch_shapes=[pltpu.VMEM((1, D_MODEL), jnp.float32)],
    )(idxf, dout_flat, vsum, wot, bo2)


@jax.jit
def kernel(x, Wq, bq, Wk, bk, Wv, bv, Wo, bo):
    wq_bf = Wq.astype(jnp.bfloat16)
    wk_bf = Wk.astype(jnp.bfloat16)
    wv_bf = Wv.astype(jnp.bfloat16)
    wo_bf = Wo.astype(jnp.bfloat16)
    b3 = jnp.stack([bq, bk, bv]).reshape(1, 3, D_MODEL)
    bo2 = bo.reshape(1, D_MODEL)

    q, k, v, sp, vsum = _qkv_stage(x, wq_bf, wk_bf, wv_bf, b3)
    top_idx = _topk_stage(sp.reshape(B * N_HEADS, S))
    idx3 = top_idx.reshape(B * N_HEADS, 1, SAMPLED)
    dout = _attn_stage(idx3, q, k, v, wo_bf, vsum)
    idxf = top_idx.reshape(B, 1, NSEL)
    dout_flat = dout.reshape(B, NSEL, D_MODEL)
    return _assemble_stage(idxf, dout_flat, vsum, wo_bf, bo2)


# speedup vs baseline: 3.3202x; 1.0047x over previous
import math

import jax
import jax.numpy as jnp
from jax.experimental import pallas as pl
from jax.experimental.pallas import tpu as pltpu

D_MODEL = 1024
N_HEADS = 16
HEAD_DIM = D_MODEL // N_HEADS
B = 2
S = 4096
SAMPLED = max(1, min(S, int(5 * math.log(S + 1))))
BS = 512
OC = 512
NSEL = N_HEADS * SAMPLED


def _qkv_body(x_ref, wq_ref, wk_ref, wv_ref, b_ref,
              q_ref, k_ref, v_ref, sp_ref, vsum_ref):
    s = pl.program_id(1)
    x = x_ref[0].astype(jnp.bfloat16)
    tdims = (((1,), (1,)), ((), ()))
    q = jax.lax.dot_general(x, wq_ref[...], tdims,
                            preferred_element_type=jnp.float32)
    q = q + b_ref[0, 0]
    q_ref[0] = q.astype(jnp.bfloat16)
    k = jax.lax.dot_general(x, wk_ref[...], tdims,
                            preferred_element_type=jnp.float32)
    k_ref[0] = (k + b_ref[0, 1]).astype(jnp.bfloat16)
    v = jax.lax.dot_general(x, wv_ref[...], tdims,
                            preferred_element_type=jnp.float32)
    v = v + b_ref[0, 2]
    v_ref[0] = v.astype(jnp.bfloat16)
    q2 = (q * q).reshape(BS, N_HEADS, HEAD_DIM)
    sp = jnp.sum(q2, axis=2)
    sp_ref[0] = sp.T

    @pl.when(s == 0)
    def _():
        vsum_ref[0] = jnp.zeros((1, D_MODEL), jnp.float32)
    vsum_ref[0] += jnp.sum(v, axis=0, keepdims=True)


def _qkv_stage(x, wq, wk, wv, b3):
    grid = (B, S // BS)
    return pl.pallas_call(
        _qkv_body,
        grid=grid,
        in_specs=[
            pl.BlockSpec((1, BS, D_MODEL), lambda b, s: (b, s, 0)),
            pl.BlockSpec((D_MODEL, D_MODEL), lambda b, s: (0, 0)),
            pl.BlockSpec((D_MODEL, D_MODEL), lambda b, s: (0, 0)),
            pl.BlockSpec((D_MODEL, D_MODEL), lambda b, s: (0, 0)),
            pl.BlockSpec((1, 3, D_MODEL), lambda b, s: (0, 0, 0)),
        ],
        out_specs=[
            pl.BlockSpec((1, BS, D_MODEL), lambda b, s: (b, s, 0)),
            pl.BlockSpec((1, BS, D_MODEL), lambda b, s: (b, s, 0)),
            pl.BlockSpec((1, BS, D_MODEL), lambda b, s: (b, s, 0)),
            pl.BlockSpec((1, N_HEADS, BS), lambda b, s: (b, 0, s)),
            pl.BlockSpec((1, 1, D_MODEL), lambda b, s: (b, 0, 0)),
        ],
        out_shape=[
            jax.ShapeDtypeStruct((B, S, D_MODEL), jnp.bfloat16),
            jax.ShapeDtypeStruct((B, S, D_MODEL), jnp.bfloat16),
            jax.ShapeDtypeStruct((B, S, D_MODEL), jnp.bfloat16),
            jax.ShapeDtypeStruct((B, N_HEADS, S), jnp.float32),
            jax.ShapeDtypeStruct((B, 1, D_MODEL), jnp.float32),
        ],
    )(x, wq, wk, wv, b3)


def _topk_body(sp_ref, idx_ref):
    vals = sp_ref[...]
    col = jax.lax.broadcasted_iota(jnp.int32, vals.shape, 1)
    neg = jnp.float32(-jnp.inf)
    for t in range(SAMPLED):
        m = jnp.max(vals, axis=1, keepdims=True)
        eq = vals == m
        arg = jnp.min(jnp.where(eq, col, S), axis=1, keepdims=True)
        idx_ref[:, t:t + 1] = arg
        vals = jnp.where(col == arg, neg, vals)


def _topk_stage(sp):
    return pl.pallas_call(
        _topk_body,
        out_shape=jax.ShapeDtypeStruct((B * N_HEADS, SAMPLED), jnp.int32),
    )(sp)


HP = 8
CW = HP * HEAD_DIM
RW = HP * SAMPLED


def _attn_body(idx_ref, q_ref, k_ref, v_ref, wo_ref, vsum_ref, dout_ref):
    srow = jax.lax.broadcasted_iota(jnp.int32, (S, RW), 0)
    idx_cat = jnp.concatenate([idx_ref[j] for j in range(HP)], axis=1)
    oht = (srow == idx_cat).astype(jnp.bfloat16)
    rblk = jax.lax.broadcasted_iota(jnp.int32, (RW, CW), 0) // SAMPLED
    cblk = jax.lax.broadcasted_iota(jnp.int32, (RW, CW), 1) // HEAD_DIM
    mask = rblk == cblk
    selqf = jax.lax.dot_general(oht, q_ref[0], (((0,), (0,)), ((), ())),
                                preferred_element_type=jnp.float32)
    selq = jnp.where(mask, selqf, 0.0).astype(jnp.bfloat16)
    scores = jax.lax.dot_general(selq, k_ref[0], (((1,), (1,)), ((), ())),
                                 preferred_element_type=jnp.float32)
    scores = scores * (HEAD_DIM ** (-0.5))
    m = jnp.max(scores, axis=1, keepdims=True)
    p = jnp.exp(scores - m)
    denom = jnp.sum(p, axis=1, keepdims=True)
    ctx = jax.lax.dot_general(p.astype(jnp.bfloat16), v_ref[0],
                              (((1,), (0,)), ((), ())),
                              preferred_element_type=jnp.float32)
    ctx = ctx / denom
    mean4 = vsum_ref[0, 0] * jnp.float32(1.0 / S)
    delta = jnp.where(mask, ctx - mean4, 0.0).astype(jnp.bfloat16)
    dout = jax.lax.dot_general(delta, wo_ref[...], (((1,), (1,)), ((), ())),
                               preferred_element_type=jnp.float32)
    dout = dout.astype(jnp.bfloat16)
    for j in range(HP):
        dout_ref[0, j] = dout[j * SAMPLED:(j + 1) * SAMPLED]


def _attn_stage(idx3, q, k, v, wot, vsum):
    grid = (B, N_HEADS // HP)
    return pl.pallas_call(
        _attn_body,
        grid=grid,
        in_specs=[
            pl.BlockSpec((HP, 1, SAMPLED),
                         lambda b, g: (b * (N_HEADS // HP) + g, 0, 0)),
            pl.BlockSpec((1, S, CW), lambda b, g: (b, 0, g)),
            pl.BlockSpec((1, S, CW), lambda b, g: (b, 0, g)),
            pl.BlockSpec((1, S, CW), lambda b, g: (b, 0, g)),
            pl.BlockSpec((D_MODEL, CW), lambda b, g: (0, g)),
            pl.BlockSpec((1, 1, CW), lambda b, g: (b, 0, g)),
        ],
        out_specs=pl.BlockSpec((1, HP, SAMPLED, D_MODEL),
                               lambda b, g: (b, g, 0, 0)),
        out_shape=jax.ShapeDtypeStruct((B, N_HEADS, SAMPLED, D_MODEL),
                                       jnp.bfloat16),
    )(idx3, q, k, v, wot, vsum)


def _assemble_body(idxf_ref, dout_ref, vsum_ref, wo_ref, bo_ref, out_ref,
                   base_ref):
    c = pl.program_id(1)

    @pl.when(c == 0)
    def _():
        mean_flat = vsum_ref[0] * jnp.float32(1.0 / S)
        base = jax.lax.dot_general(mean_flat.astype(jnp.bfloat16),
                                   wo_ref[...], (((1,), (1,)), ((), ())),
                                   preferred_element_type=jnp.float32)
        base_ref[...] = base + bo_ref[...]

    row = jax.lax.broadcasted_iota(jnp.int32, (OC, NSEL), 0) + c * OC
    oh = (row == idxf_ref[0]).astype(jnp.bfloat16)
    scat = jax.lax.dot_general(oh, dout_ref[0], (((1,), (0,)), ((), ())),
                               preferred_element_type=jnp.float32)
    out_ref[0] = scat + base_ref[...]


def _assemble_stage(idxf, dout_flat, vsum, wot, bo2):
    grid = (B, S // OC)
    return pl.pallas_call(
        _assemble_body,
        grid=grid,
        in_specs=[
            pl.BlockSpec((1, 1, NSEL), lambda b, c: (b, 0, 0)),
            pl.BlockSpec((1, NSEL, D_MODEL), lambda b, c: (b, 0, 0)),
            pl.BlockSpec((1, 1, D_MODEL), lambda b, c: (b, 0, 0)),
            pl.BlockSpec((D_MODEL, D_MODEL), lambda b, c: (0, 0)),
            pl.BlockSpec((1, D_MODEL), lambda b, c: (0, 0)),
        ],
        out_specs=pl.BlockSpec((1, OC, D_MODEL), lambda b, c: (b, c, 0)),
        out_shape=jax.ShapeDtypeStruct((B, S, D_MODEL), jnp.float32),
        scratch_shapes=[pltpu.VMEM((1, D_MODEL), jnp.float32)],
    )(idxf, dout_flat, vsum, wot, bo2)


@jax.jit
def kernel(x, Wq, bq, Wk, bk, Wv, bv, Wo, bo):
    wq_bf = Wq.astype(jnp.bfloat16)
    wk_bf = Wk.astype(jnp.bfloat16)
    wv_bf = Wv.astype(jnp.bfloat16)
    wo_bf = Wo.astype(jnp.bfloat16)
    b3 = jnp.stack([bq, bk, bv]).reshape(1, 3, D_MODEL)
    bo2 = bo.reshape(1, D_MODEL)

    q, k, v, sp, vsum = _qkv_stage(x, wq_bf, wk_bf, wv_bf, b3)
    top_idx = _topk_stage(sp.reshape(B * N_HEADS, S))
    idx3 = top_idx.reshape(B * N_HEADS, 1, SAMPLED)
    dout = _attn_stage(idx3, q, k, v, wo_bf, vsum)
    idxf = top_idx.reshape(B, 1, NSEL)
    dout_flat = dout.reshape(B, NSEL, D_MODEL)
    return _assemble_stage(idxf, dout_flat, vsum, wo_bf, bo2)
